# Initial kernel scaffold; baseline (speedup 1.0000x reference)
import jax
import jax.numpy as jnp
from jax.experimental import pallas as pl


def kernel(logits, targets):
    raise NotImplementedError("write your pallas kernel here")


# speedup vs baseline: 200.0227x; 200.0227x over previous
import functools

import jax
import jax.numpy as jnp
from jax import lax
from jax.experimental import pallas as pl
from jax.experimental.pallas import tpu as pltpu
from jax.experimental.pallas import tpu_sc as plsc

K = 2048
LANES = 16
NB = 2 * K * LANES
H = 512
W = 512
HW = H * W
B = 4
NPIX = B * HW
NCLS = 20
CH = 16384
ROWBLK = 16
WBLK = 128


def _prep_body(x_ref, t_ref, o_ref):
    x = x_ref[0]
    m = jnp.max(x, axis=0, keepdims=True)
    ex = jnp.exp(x - m)
    p = ex * (1.0 / jnp.sum(ex, axis=0, keepdims=True))
    t = t_ref[0]
    cls = lax.broadcasted_iota(jnp.int32, (21, ROWBLK, W), 0)
    fg = t[None, :, :] == cls
    e = jnp.abs(p - fg.astype(jnp.float32))
    k = jnp.minimum((e * K).astype(jnp.int32), K - 1)
    lane = lax.broadcasted_iota(jnp.int32, (21, ROWBLK, W), 2) % LANES
    sidx = (k + jnp.where(fg, K, 0)) * LANES + lane
    s20 = sidx[1:]
    packed = jnp.bitwise_or(
        s20[:, : ROWBLK // 2], lax.shift_left(s20[:, ROWBLK // 2 :], 16)
    )
    quarters = jnp.stack(
        [packed[:, :, q * WBLK : (q + 1) * WBLK] for q in range(W // WBLK)], axis=1
    )
    o_ref[...] = quarters[:, None]


_BSPLIT = 1


def _make_prep(b0):
    nh = H // ROWBLK
    return pl.pallas_call(
        _prep_body,
        grid=(_BSPLIT, nh),
        in_specs=[
            pl.BlockSpec((1, 21, ROWBLK, W), lambda b, h, b0=b0: (b + b0, 0, h, 0)),
            pl.BlockSpec((1, ROWBLK, W), lambda b, h, b0=b0: (b + b0, h, 0)),
        ],
        out_specs=pl.BlockSpec(
            (NCLS, 1, W // WBLK, ROWBLK // 2, WBLK),
            lambda b, h: (0, b * nh + h, 0, 0, 0),
        ),
        out_shape=jax.ShapeDtypeStruct(
            (NCLS, _BSPLIT * nh, W // WBLK, ROWBLK // 2, WBLK), jnp.int32
        ),
    )


_prep_halves = [_make_prep(b0) for b0 in range(0, B, _BSPLIT)]


_mesh = plsc.VectorSubcoreMesh(
    core_axis_name="c", subcore_axis_name="s", num_cores=2, num_subcores=16
)
_NWORDS = (_BSPLIT * HW) // 2


@functools.partial(
    pl.kernel,
    out_type=jax.ShapeDtypeStruct((NCLS * NB,), jnp.int32),
    mesh=_mesh,
    compiler_params=pltpu.CompilerParams(needs_layout_passes=False),
    scratch_types=[
        pltpu.VMEM((NB,), jnp.int32),
        pltpu.VMEM((CH,), jnp.int32),
        pltpu.VMEM((CH,), jnp.int32),
        pltpu.SemaphoreType.DMA,
        pltpu.SemaphoreType.DMA,
    ],
)
def _hist_kernel(sidx_hbm, out_hbm, hist, buf0, buf1, sem0, sem1):
    wid = lax.axis_index("s") * 2 + lax.axis_index("c")

    @pl.when(wid < NCLS)
    def _():
        zeros = jnp.zeros((LANES,), jnp.int32)

        def zbody(i, carry):
            hist[pl.ds(i * LANES, LANES)] = zeros
            return carry

        lax.fori_loop(0, NB // LANES, zbody, 0, unroll=8)

        ones = jnp.ones((LANES,), jnp.int32)
        GRP = 8

        def process(buf):
            def ibody(i, carry):
                base = i * (GRP * LANES)
                vs = [buf[pl.ds(base + j * LANES, LANES)] for j in range(GRP)]
                ks = [lax.bitwise_and(v, 0xFFFF) for v in vs]
                ks += [lax.shift_right_logical(v, 16) for v in vs]
                for kk in ks:
                    plsc.addupdate_scatter(hist, [kk], ones)
                return carry

            lax.fori_loop(0, CH // (GRP * LANES), ibody, 0)

        bufs = (buf0, buf1)
        sems = (sem0, sem1)

        def chunk_src(ci):
            return sidx_hbm.at[pl.ds(wid * _NWORDS + ci * CH, CH)]

        pending = pltpu.async_copy(chunk_src(0), bufs[0], sems[0])
        for ci in range(_NWORDS // CH):
            nxt = None
            if ci + 1 < _NWORDS // CH:
                nxt = pltpu.async_copy(
                    chunk_src(ci + 1), bufs[(ci + 1) % 2], sems[(ci + 1) % 2]
                )
            pending.wait()
            process(bufs[ci % 2])
            pending = nxt

        pltpu.sync_copy(hist, out_hbm.at[pl.ds(wid * NB, NB)])


_GROUPS = 128 // LANES
_ROWS = NB // 128


_NHALVES = B // _BSPLIT


def _drain_body(*refs):
    o_ref = refs[-1]
    acc = refs[0][...]
    for r in refs[1:-1]:
        acc = acc + r[...]
    h = acc.astype(jnp.float32)
    lane = lax.broadcasted_iota(jnp.int32, (128, _GROUPS), 0)
    grp = lax.broadcasted_iota(jnp.int32, (128, _GROUPS), 1)
    S = ((lane // LANES) == grp).astype(jnp.float32)
    g = lax.dot_general(h, S, (((2,), (0,)), ((), ())))
    n = g[:, : _ROWS // 2, :]
    f = g[:, _ROWS // 2 :, :]

    R = _ROWS // 2
    rp = lax.broadcasted_iota(jnp.int32, (R, R), 0)
    rq = lax.broadcasted_iota(jnp.int32, (R, R), 1)
    t_strict = (rp < rq).astype(jnp.float32)
    jp = lax.broadcasted_iota(jnp.int32, (_GROUPS, _GROUPS), 0)
    jq = lax.broadcasted_iota(jnp.int32, (_GROUPS, _GROUPS), 1)
    u_incl = (jp <= jq).astype(jnp.float32)

    def cumflat(x):
        rowsum = jnp.sum(x, axis=2)
        rowpref = lax.dot_general(rowsum, t_strict, (((1,), (0,)), ((), ())))
        inner = lax.dot_general(x, u_incl, (((2,), (0,)), ((), ())))
        return rowpref[:, :, None] + inner

    cum_f = cumflat(f)
    cum_n = cumflat(n) + cum_f
    npix = jnp.float32(NPIX)
    terms = 1.0 - cum_f / jnp.maximum(cum_f + (npix - cum_n), 1.0)
    lp = jnp.sum(terms, axis=(1, 2)) / K + 0.5 / K

    ntot = n + f
    p_cnt = jnp.sum(f, axis=(1, 2))
    r_i = lax.broadcasted_iota(jnp.int32, (R, _GROUPS), 0)
    j_i = lax.broadcasted_iota(jnp.int32, (R, _GROUPS), 1)
    k2 = r_i * _GROUPS + j_i
    ehat = (k2.astype(jnp.float32) + 0.5) / K
    sum_e = jnp.sum(ntot * ehat[None], axis=(1, 2))
    maxk = jnp.max(jnp.where(ntot > 0, k2[None], -1), axis=(1, 2))
    max_e = (maxk.astype(jnp.float32) + 0.5) / K

    present = p_cnt > 0
    high = max_e > 0.1
    la = jnp.where(high, sum_e / npix, 0.0)
    losses = jnp.where(present, lp, la)
    wts = jnp.logical_or(present, high).astype(jnp.float32)
    d = jnp.sum(wts)
    out = jnp.where(d > 0, jnp.sum(losses * wts) / jnp.maximum(d, 1.0), 0.0)
    o_ref[0, 0] = out


_drain = pl.pallas_call(
    _drain_body,
    in_specs=[
        pl.BlockSpec((NCLS, _ROWS, 128), lambda: (0, 0, 0))
        for _ in range(_NHALVES)
    ],
    out_specs=pl.BlockSpec(memory_space=pltpu.SMEM),
    out_shape=jax.ShapeDtypeStruct((1, 1), jnp.float32),
)


def kernel(logits, targets):
    halves = []
    for prep in _prep_halves:
        sidx = prep(logits, targets)
        sflat = sidx.reshape(NCLS * _NWORDS)
        hists = _hist_kernel(sflat)
        halves.append(hists.reshape(NCLS, _ROWS, 128))
    out = _drain(*halves)
    return out[0, 0]
